# baseline (speedup 1.0000x reference)
import functools

import jax
import jax.numpy as jnp
from jax import lax
from jax.experimental import pallas as pl
from jax.experimental.pallas import tpu as pltpu
from jax.experimental.pallas import tpu_sc as plsc


def _proj_body(WT_ref, bcol_ref, T_ref, out_ref):
    out_ref[...] = lax.dot_general(
        WT_ref[...], T_ref[...], (((1,), (0,)), ((), ())),
        precision=jax.lax.Precision.HIGHEST) + bcol_ref[...]


def _project_table(tableT, WT, bcol):
    D, V = tableT.shape
    C = WT.shape[0]
    CHUNK = 8192
    grid = (V + CHUNK - 1) // CHUNK
    return pl.pallas_call(
        _proj_body,
        grid=(grid,),
        in_specs=[
            pl.BlockSpec((C, D), lambda i: (0, 0)),
            pl.BlockSpec((C, 1), lambda i: (0, 0)),
            pl.BlockSpec((D, CHUNK), lambda i: (0, i)),
        ],
        out_specs=pl.BlockSpec((C, CHUNK), lambda i: (0, i)),
        out_shape=jax.ShapeDtypeStruct((C, V), jnp.float32),
        compiler_params=pltpu.CompilerParams(
            dimension_semantics=("arbitrary",)),
    )(WT, bcol, tableT)


def _sc_gather(ptable128, idx, C, B, L):
    N = idx.shape[0]
    assert B & (B - 1) == 0
    bsh = B.bit_length() - 1
    info = plsc.get_sparse_core_info()
    NC, NS, LN = info.num_cores, info.num_subcores, info.num_lanes
    NW = NC * NS
    assert N % (NW * 8) == 0
    per_w = N // NW
    K = 128
    assert per_w % K == 0
    n_chunks = per_w // K
    NBUF = 5 if n_chunks % 5 == 0 else (4 if n_chunks % 4 == 0 else 2)
    assert n_chunks % NBUF == 0
    n_groups = n_chunks // NBUF
    PK = C * 8

    mesh = plsc.VectorSubcoreMesh(core_axis_name="c", subcore_axis_name="s")

    @functools.partial(
        pl.kernel,
        mesh=mesh,
        compiler_params=pltpu.CompilerParams(use_tc_tiling_on_sc=False),
        out_type=[jax.ShapeDtypeStruct((N // 8, PK), jnp.float32),
                  jax.ShapeDtypeStruct((N, C), jnp.float32)],
        scratch_types=(
            [pltpu.VMEM((per_w,), jnp.int32),
             pltpu.VMEM((per_w,), jnp.int32),
             pltpu.VMEM((per_w,), jnp.int32),
             pltpu.VMEM((n_chunks, K), jnp.int32)]
            + [pltpu.VMEM((K, PK), jnp.float32) for _ in range(NBUF)]
            + [pltpu.VMEM((K // 8, PK), jnp.float32) for _ in range(NBUF)]
            + [pltpu.VMEM((K, C), jnp.float32) for _ in range(NBUF)]
            + [pltpu.SemaphoreType.DMA for _ in range(3 * NBUF)]
        ),
    )
    def k(tab_hbm, idx_hbm, out_hbm, out2_hbm, *scr):
        idx_v, idxu_v, offs_v, pos_v = scr[0], scr[1], scr[2], scr[3]
        bufs = scr[4:4 + NBUF]
        obufs = scr[4 + NBUF:4 + 2 * NBUF]
        obufs2 = scr[4 + 2 * NBUF:4 + 3 * NBUF]
        gsems = scr[4 + 3 * NBUF:4 + 4 * NBUF]
        wsems = scr[4 + 4 * NBUF:4 + 5 * NBUF]
        ssems = scr[4 + 5 * NBUF:4 + 6 * NBUF]
        wid = lax.axis_index("s") * NC + lax.axis_index("c")
        base = pl.multiple_of(wid * per_w, 128)
        obase = pl.multiple_of(wid * (per_w // 8), 16)
        pltpu.sync_copy(idx_hbm.at[pl.ds(base, per_w)], idx_v)

        def prep(i, carry):
            v = idx_v[pl.ds(i * LN, LN)]
            idxu_v[pl.ds(i * LN, LN)] = lax.shift_right_logical(v, 3)
            offs_v[pl.ds(i * LN, LN)] = (v & 7) * C
            n = base + i * LN + lax.iota(jnp.int32, LN)
            pos_v[lax.shift_right_logical(i, 3),
                  pl.ds((i & 7) * LN, LN)] = (
                (n & (B - 1)) * L + lax.shift_right_logical(n, bsh))
            return carry
        lax.fori_loop(0, per_w // LN, prep, 0, unroll=False)

        def group(g, carry):
            off = g * (NBUF * K)
            gathers = []
            for b in range(NBUF):
                cp = pltpu.make_async_copy(
                    tab_hbm.at[idxu_v.at[pl.ds(off + b * K, K)]],
                    bufs[b], gsems[b])
                cp.start()
                gathers.append(cp)
            for b in range(NBUF):
                gathers[b].wait()
                def sel(q, c2):
                    tk = q * LN
                    sv = offs_v[pl.ds(off + b * K + tk, LN)]
                    for i in range(LN):
                        kk = tk + i
                        vals = bufs[b][kk, pl.ds(sv[i], C)]
                        obufs[b][lax.shift_right_logical(kk, 3),
                                 pl.ds((kk & 7) * C, C)] = vals
                        obufs2[b][kk, pl.ds(0, C)] = vals
                    return c2
                lax.fori_loop(0, K // LN, sel, 0, unroll=False)
                pltpu.make_async_copy(
                    obufs[b],
                    out_hbm.at[pl.ds(
                        pl.multiple_of(obase + (off + b * K) // 8, 16),
                        K // 8)],
                    wsems[b]).start()
                pltpu.make_async_copy(
                    obufs2[b],
                    out2_hbm.at[pos_v.at[g * NBUF + b]],
                    ssems[b]).start()
            for b in range(NBUF):
                pltpu.make_async_copy(
                    obufs[b],
                    out_hbm.at[pl.ds(
                        pl.multiple_of(obase + (off + b * K) // 8, 16),
                        K // 8)],
                    wsems[b]).wait()
                pltpu.make_async_copy(
                    obufs2[b],
                    out2_hbm.at[pos_v.at[g * NBUF + b]],
                    ssems[b]).wait()
            return carry

        lax.fori_loop(0, n_groups, group, 0, unroll=False)

    return k(ptable128, idx)


def _gmax(x, lanemod0, bd1, HI):
    r = x
    for s in (1, 2, 4, 8):
        r = jnp.maximum(r, pltpu.roll(r, 128 - s, 1))
    rm = jnp.where(lanemod0, r, 0.0)
    return lax.dot_general(rm, bd1, (((1,), (0,)), ((), ())), precision=HI)


def _crf_body(L, C, em_hbm, lab_ref, trans_ref, start_ref, end_ref,
              loss_ref, alpha, gold, ohprev, bd1_s, bdexp_s, bdraw_s, se_s,
              eb0, eb1, sem0, sem1):
    t = pl.program_id(0)
    HI = jax.lax.Precision.HIGHEST
    par = t % 2

    def dma(tt, buf, sem):
        return pltpu.make_async_copy(
            em_hbm.at[pl.ds(tt * 128, 128)], buf, sem)

    @pl.when(t == 0)
    def _():
        dma(0, eb0, sem0).start()

    @pl.when(par == 0)
    def _():
        dma(t, eb0, sem0).wait()

        @pl.when(t + 1 < L)
        def _():
            dma(t + 1, eb1, sem1).start()

    @pl.when(par == 1)
    def _():
        dma(t, eb1, sem1).wait()

        @pl.when(t + 1 < L)
        def _():
            dma(t + 1, eb0, sem0).start()

    E = eb0[...] if L == 1 else lax.cond(
        par == 0, lambda: eb0[...], lambda: eb1[...])
    lab = lab_ref[0]
    i1 = lax.broadcasted_iota(jnp.int32, (128, 128), 1)
    lanemod0 = (i1 & (C - 1)) == 0
    oh = (lab == (i1 & (C - 1))).astype(jnp.float32)

    @pl.when(t == 0)
    def _():
        i0 = lax.broadcasted_iota(jnp.int32, (128, 128), 0)
        bd1 = (lax.shift_right_logical(i0, 4)
               == lax.shift_right_logical(i1, 4)).astype(jnp.float32)
        A = ((lax.broadcasted_iota(jnp.int32, (128, C), 0) & (C - 1))
             == lax.broadcasted_iota(jnp.int32, (128, C), 1)
             ).astype(jnp.float32)
        B16 = (lax.broadcasted_iota(jnp.int32, (C, 128), 0)
               == (lax.broadcasted_iota(jnp.int32, (C, 128), 1) & (C - 1))
               ).astype(jnp.float32)
        tt = lax.dot_general(
            lax.dot_general(A, trans_ref[...], (((1,), (0,)), ((), ())),
                            precision=HI),
            B16, (((1,), (0,)), ((), ())), precision=HI)
        bd1_s[...] = bd1
        bdexp_s[...] = jnp.exp(tt) * bd1
        bdraw_s[...] = tt * bd1
        strow = lax.dot_general(start_ref[...], B16, (((0,), (0,)), ((), ())),
                                precision=HI)
        endrow = lax.dot_general(end_ref[...], B16, (((0,), (0,)), ((), ())),
                                 precision=HI)
        se_s[0:1, :] = strow
        se_s[1:2, :] = endrow
        alpha[...] = strow + E
        gold[...] = oh * (strow + E)
        ohprev[...] = oh
        loss_ref[...] = jnp.zeros((1, 1), jnp.float32)

    @pl.when(t > 0)
    def _():
        bd1 = bd1_s[...]
        a = alpha[...]
        M = _gmax(a, lanemod0, bd1, HI)
        ea = jnp.exp(a - M)
        S = lax.dot_general(ea, bdexp_s[...], (((1,), (0,)), ((), ())),
                            precision=HI)
        alpha[...] = M + jnp.log(S) + E
        gtr = lax.dot_general(ohprev[...], bdraw_s[...],
                              (((1,), (0,)), ((), ())), precision=HI)
        gold[...] = gold[...] + oh * (E + gtr)
        ohprev[...] = oh

    @pl.when(t == L - 1)
    def _():
        bd1 = bd1_s[...]
        endrow = se_s[1:2, :]
        a2 = alpha[...] + endrow
        M2 = _gmax(a2, lanemod0, bd1, HI)
        S2 = lax.dot_general(jnp.exp(a2 - M2), bd1, (((1,), (0,)), ((), ())),
                             precision=HI)
        logZ = jnp.where(lanemod0, M2 + jnp.log(S2), 0.0)
        goldtot = gold[...] + oh * endrow
        g1 = jnp.sum(goldtot - logZ, axis=0, keepdims=True)
        val = jnp.sum(g1, axis=1, keepdims=True)
        loss_ref[...] = -val


def _crf_call(em_p, lab_p3, trans, start2, end2, interpret=False):
    L = lab_p3.shape[0]
    C = trans.shape[0]
    body = functools.partial(_crf_body, L, C)
    loss = pl.pallas_call(
        body,
        grid=(L,),
        in_specs=[
            pl.BlockSpec(memory_space=pl.ANY),
            pl.BlockSpec((1, 128, 128), lambda l: (l, 0, 0)),
            pl.BlockSpec((C, C), lambda l: (0, 0)),
            pl.BlockSpec((C, 1), lambda l: (0, 0)),
            pl.BlockSpec((C, 1), lambda l: (0, 0)),
        ],
        out_specs=pl.BlockSpec((1, 1), lambda l: (0, 0)),
        out_shape=jax.ShapeDtypeStruct((1, 1), jnp.float32),
        scratch_shapes=[
            pltpu.VMEM((128, 128), jnp.float32),
            pltpu.VMEM((128, 128), jnp.float32),
            pltpu.VMEM((128, 128), jnp.float32),
            pltpu.VMEM((128, 128), jnp.float32),
            pltpu.VMEM((128, 128), jnp.float32),
            pltpu.VMEM((128, 128), jnp.float32),
            pltpu.VMEM((2, 128), jnp.float32),
            pltpu.VMEM((128, 128), jnp.float32),
            pltpu.VMEM((128, 128), jnp.float32),
            pltpu.SemaphoreType.DMA,
            pltpu.SemaphoreType.DMA,
        ],
        compiler_params=pltpu.CompilerParams(
            dimension_semantics=("arbitrary",)),
        interpret=interpret,
    )(em_p, lab_p3, trans, start2, end2)
    return loss


def kernel(x, labels, table, W_fc, b_fc, start_t, end_t, trans):
    B, L = x.shape
    V, D = table.shape
    C = W_fc.shape[1]
    tableT = jnp.swapaxes(table, 0, 1)
    ptableT = _project_table(tableT, jnp.swapaxes(W_fc, 0, 1),
                             b_fc.reshape(C, 1))
    ptable128 = jnp.swapaxes(ptableT, 0, 1).reshape(V // 8, 8 * C)
    xT = jnp.swapaxes(x, 0, 1).reshape(-1)
    em_p, logits_bl = _sc_gather(ptable128, xT, C, B, L)
    labels_T = jnp.swapaxes(labels, 0, 1)
    lab_p3 = jnp.repeat(labels_T.reshape(L, B // 8, 8), C, axis=2)
    loss = _crf_call(em_p, lab_p3, trans,
                     start_t.reshape(C, 1), end_t.reshape(C, 1))
    logits = logits_bl.reshape(B, L, C)
    return (logits, loss[0, 0])


# speedup vs baseline: 1.0119x; 1.0119x over previous
import functools

import jax
import jax.numpy as jnp
from jax import lax
from jax.experimental import pallas as pl
from jax.experimental.pallas import tpu as pltpu
from jax.experimental.pallas import tpu_sc as plsc


def _proj_body(WT_ref, bcol_ref, T_ref, out_ref):
    out_ref[...] = lax.dot_general(
        WT_ref[...], T_ref[...], (((1,), (0,)), ((), ())),
        precision=jax.lax.Precision.HIGHEST) + bcol_ref[...]


def _project_table(tableT, WT, bcol):
    D, V = tableT.shape
    C = WT.shape[0]
    CHUNK = 8192
    grid = (V + CHUNK - 1) // CHUNK
    return pl.pallas_call(
        _proj_body,
        grid=(grid,),
        in_specs=[
            pl.BlockSpec((C, D), lambda i: (0, 0)),
            pl.BlockSpec((C, 1), lambda i: (0, 0)),
            pl.BlockSpec((D, CHUNK), lambda i: (0, i)),
        ],
        out_specs=pl.BlockSpec((C, CHUNK), lambda i: (0, i)),
        out_shape=jax.ShapeDtypeStruct((C, V), jnp.float32),
        compiler_params=pltpu.CompilerParams(
            dimension_semantics=("arbitrary",)),
    )(WT, bcol, tableT)


def _sc_gather(ptable128, idx, C):
    N = idx.shape[0]
    info = plsc.get_sparse_core_info()
    NC, NS, LN = info.num_cores, info.num_subcores, info.num_lanes
    NW = NC * NS
    assert N % (NW * 8) == 0
    per_w = N // NW
    K = 128
    assert per_w % K == 0
    n_chunks = per_w // K
    NBUF = 5 if n_chunks % 5 == 0 else (4 if n_chunks % 4 == 0 else 2)
    assert n_chunks % NBUF == 0
    n_groups = n_chunks // NBUF
    PK = C * 8

    mesh = plsc.VectorSubcoreMesh(core_axis_name="c", subcore_axis_name="s")

    @functools.partial(
        pl.kernel,
        mesh=mesh,
        compiler_params=pltpu.CompilerParams(use_tc_tiling_on_sc=False),
        out_type=jax.ShapeDtypeStruct((N // 8, PK), jnp.float32),
        scratch_types=(
            [pltpu.VMEM((per_w,), jnp.int32),
             pltpu.VMEM((per_w,), jnp.int32),
             pltpu.VMEM((per_w,), jnp.int32)]
            + [pltpu.VMEM((K, PK), jnp.float32) for _ in range(NBUF)]
            + [pltpu.VMEM((K // 8, PK), jnp.float32) for _ in range(NBUF)]
            + [pltpu.SemaphoreType.DMA for _ in range(2 * NBUF)]
        ),
    )
    def k(tab_hbm, idx_hbm, out_hbm, *scr):
        idx_v, idxu_v, offs_v = scr[0], scr[1], scr[2]
        bufs = scr[3:3 + NBUF]
        obufs = scr[3 + NBUF:3 + 2 * NBUF]
        gsems = scr[3 + 2 * NBUF:3 + 3 * NBUF]
        wsems = scr[3 + 3 * NBUF:3 + 4 * NBUF]
        wid = lax.axis_index("s") * NC + lax.axis_index("c")
        base = pl.multiple_of(wid * per_w, 128)
        obase = pl.multiple_of(wid * (per_w // 8), 16)
        pltpu.sync_copy(idx_hbm.at[pl.ds(base, per_w)], idx_v)

        def prep(i, carry):
            v = idx_v[pl.ds(i * LN, LN)]
            idxu_v[pl.ds(i * LN, LN)] = lax.shift_right_logical(v, 3)
            offs_v[pl.ds(i * LN, LN)] = (v & 7) * C
            return carry
        lax.fori_loop(0, per_w // LN, prep, 0, unroll=False)

        def group(g, carry):
            off = g * (NBUF * K)
            gathers = []
            for b in range(NBUF):
                cp = pltpu.make_async_copy(
                    tab_hbm.at[idxu_v.at[pl.ds(off + b * K, K)]],
                    bufs[b], gsems[b])
                cp.start()
                gathers.append(cp)
            writes = []
            for b in range(NBUF):
                gathers[b].wait()
                def sel(q, c2):
                    tk = q * LN
                    sv = offs_v[pl.ds(off + b * K + tk, LN)]
                    for i in range(LN):
                        kk = tk + i
                        vals = bufs[b][kk, pl.ds(sv[i], C)]
                        obufs[b][lax.shift_right_logical(kk, 3),
                                 pl.ds((kk & 7) * C, C)] = vals
                    return c2
                lax.fori_loop(0, K // LN, sel, 0, unroll=False)
                wp = pltpu.make_async_copy(
                    obufs[b],
                    out_hbm.at[pl.ds(
                        pl.multiple_of(obase + (off + b * K) // 8, 16),
                        K // 8)],
                    wsems[b])
                wp.start()
                writes.append(wp)
            for b in range(NBUF):
                writes[b].wait()
            return carry

        lax.fori_loop(0, n_groups, group, 0, unroll=False)

    return k(ptable128, idx)


def _gmax(x, lanemod0, bd1, HI):
    r = x
    for s in (1, 2, 4, 8):
        r = jnp.maximum(r, pltpu.roll(r, 128 - s, 1))
    rm = jnp.where(lanemod0, r, 0.0)
    return lax.dot_general(rm, bd1, (((1,), (0,)), ((), ())), precision=HI)


def _crf_body(L, C, em_ref, lab_ref, trans_ref, start_ref, end_ref,
              loss_ref, alpha, gold, ohprev, bd1_s, bdexp_s, bdraw_s, se_s):
    t = pl.program_id(0)
    HI = jax.lax.Precision.HIGHEST
    E = em_ref[0]
    lab = lab_ref[0]
    i1 = lax.broadcasted_iota(jnp.int32, (128, 128), 1)
    lanemod0 = (i1 & (C - 1)) == 0
    oh = (lab == (i1 & (C - 1))).astype(jnp.float32)

    @pl.when(t == 0)
    def _():
        i0 = lax.broadcasted_iota(jnp.int32, (128, 128), 0)
        bd1 = (lax.shift_right_logical(i0, 4)
               == lax.shift_right_logical(i1, 4)).astype(jnp.float32)
        A = ((lax.broadcasted_iota(jnp.int32, (128, C), 0) & (C - 1))
             == lax.broadcasted_iota(jnp.int32, (128, C), 1)
             ).astype(jnp.float32)
        B16 = (lax.broadcasted_iota(jnp.int32, (C, 128), 0)
               == (lax.broadcasted_iota(jnp.int32, (C, 128), 1) & (C - 1))
               ).astype(jnp.float32)
        tt = lax.dot_general(
            lax.dot_general(A, trans_ref[...], (((1,), (0,)), ((), ())),
                            precision=HI),
            B16, (((1,), (0,)), ((), ())), precision=HI)
        bd1_s[...] = bd1
        bdexp_s[...] = jnp.exp(tt) * bd1
        bdraw_s[...] = tt * bd1
        strow = lax.dot_general(start_ref[...], B16, (((0,), (0,)), ((), ())),
                                precision=HI)
        endrow = lax.dot_general(end_ref[...], B16, (((0,), (0,)), ((), ())),
                                 precision=HI)
        se_s[0:1, :] = strow
        se_s[1:2, :] = endrow
        alpha[...] = strow + E
        gold[...] = oh * (strow + E)
        ohprev[...] = oh
        loss_ref[...] = jnp.zeros((1, 1), jnp.float32)

    @pl.when(t > 0)
    def _():
        bd1 = bd1_s[...]
        a = alpha[...]
        M = _gmax(a, lanemod0, bd1, HI)
        ea = jnp.exp(a - M)
        S = lax.dot_general(ea, bdexp_s[...], (((1,), (0,)), ((), ())),
                            precision=HI)
        alpha[...] = M + jnp.log(S) + E
        gtr = lax.dot_general(ohprev[...], bdraw_s[...],
                              (((1,), (0,)), ((), ())), precision=HI)
        gold[...] = gold[...] + oh * (E + gtr)
        ohprev[...] = oh

    @pl.when(t == L - 1)
    def _():
        bd1 = bd1_s[...]
        endrow = se_s[1:2, :]
        a2 = alpha[...] + endrow
        M2 = _gmax(a2, lanemod0, bd1, HI)
        S2 = lax.dot_general(jnp.exp(a2 - M2), bd1, (((1,), (0,)), ((), ())),
                             precision=HI)
        logZ = jnp.where(lanemod0, M2 + jnp.log(S2), 0.0)
        goldtot = gold[...] + oh * endrow
        g1 = jnp.sum(goldtot - logZ, axis=0, keepdims=True)
        val = jnp.sum(g1, axis=1, keepdims=True)
        loss_ref[...] = -val


def _crf_call(em_p3, lab_p3, trans, start2, end2, interpret=False):
    L = em_p3.shape[0]
    C = trans.shape[0]
    body = functools.partial(_crf_body, L, C)
    loss = pl.pallas_call(
        body,
        grid=(L,),
        in_specs=[
            pl.BlockSpec((1, 128, 128), lambda l: (l, 0, 0)),
            pl.BlockSpec((1, 128, 128), lambda l: (l, 0, 0)),
            pl.BlockSpec((C, C), lambda l: (0, 0)),
            pl.BlockSpec((C, 1), lambda l: (0, 0)),
            pl.BlockSpec((C, 1), lambda l: (0, 0)),
        ],
        out_specs=pl.BlockSpec((1, 1), lambda l: (0, 0)),
        out_shape=jax.ShapeDtypeStruct((1, 1), jnp.float32),
        scratch_shapes=[
            pltpu.VMEM((128, 128), jnp.float32),
            pltpu.VMEM((128, 128), jnp.float32),
            pltpu.VMEM((128, 128), jnp.float32),
            pltpu.VMEM((128, 128), jnp.float32),
            pltpu.VMEM((128, 128), jnp.float32),
            pltpu.VMEM((128, 128), jnp.float32),
            pltpu.VMEM((2, 128), jnp.float32),
        ],
        compiler_params=pltpu.CompilerParams(
            dimension_semantics=("arbitrary",)),
        interpret=interpret,
    )(em_p3, lab_p3, trans, start2, end2)
    return loss


def kernel(x, labels, table, W_fc, b_fc, start_t, end_t, trans):
    B, L = x.shape
    V, D = table.shape
    C = W_fc.shape[1]
    tableT = jnp.swapaxes(table, 0, 1)
    ptableT = _project_table(tableT, jnp.swapaxes(W_fc, 0, 1),
                             b_fc.reshape(C, 1))
    ptable128 = lax.reshape(ptableT, (V // 8, 8 * C), dimensions=(1, 0))
    xT = jnp.swapaxes(x, 0, 1).reshape(-1)
    em_p = _sc_gather(ptable128, xT, C)
    em_p3 = em_p.reshape(L, B // 8, 8 * C)
    labels_T = jnp.swapaxes(labels, 0, 1)
    lab_p3 = jnp.repeat(labels_T.reshape(L, B // 8, 8), C, axis=2)
    loss = _crf_call(em_p3, lab_p3, trans,
                     start_t.reshape(C, 1), end_t.reshape(C, 1))
    logits = jnp.swapaxes(em_p.reshape(L, B, C), 0, 1)
    return (logits, loss[0, 0])


# speedup vs baseline: 1.0209x; 1.0089x over previous
import functools

import jax
import jax.numpy as jnp
from jax import lax
from jax.experimental import pallas as pl
from jax.experimental.pallas import tpu as pltpu
from jax.experimental.pallas import tpu_sc as plsc


def _proj_body(WT_ref, bcol_ref, T_ref, out_ref):
    out_ref[...] = lax.dot_general(
        WT_ref[...], T_ref[...], (((1,), (0,)), ((), ())),
        precision=jax.lax.Precision.HIGHEST) + bcol_ref[...]


def _project_table(tableT, WT, bcol):
    D, V = tableT.shape
    C = WT.shape[0]
    CHUNK = 8192
    grid = (V + CHUNK - 1) // CHUNK
    return pl.pallas_call(
        _proj_body,
        grid=(grid,),
        in_specs=[
            pl.BlockSpec((C, D), lambda i: (0, 0)),
            pl.BlockSpec((C, 1), lambda i: (0, 0)),
            pl.BlockSpec((D, CHUNK), lambda i: (0, i)),
        ],
        out_specs=pl.BlockSpec((C, CHUNK), lambda i: (0, i)),
        out_shape=jax.ShapeDtypeStruct((C, V), jnp.float32),
        compiler_params=pltpu.CompilerParams(
            dimension_semantics=("arbitrary",)),
    )(WT, bcol, tableT)


def _sc_gather(ptable128, idx, C):
    N = idx.shape[0]
    info = plsc.get_sparse_core_info()
    NC, NS, LN = info.num_cores, info.num_subcores, info.num_lanes
    NW = NC * NS
    assert N % (NW * 8) == 0
    per_w = N // NW
    K = 128
    assert per_w % K == 0
    n_chunks = per_w // K
    NBUF = 5 if n_chunks % 5 == 0 else (4 if n_chunks % 4 == 0 else 2)
    assert n_chunks % NBUF == 0
    n_groups = n_chunks // NBUF
    PK = C * 8

    mesh = plsc.VectorSubcoreMesh(core_axis_name="c", subcore_axis_name="s")

    @functools.partial(
        pl.kernel,
        mesh=mesh,
        compiler_params=pltpu.CompilerParams(use_tc_tiling_on_sc=False),
        out_type=jax.ShapeDtypeStruct((N // 8, PK), jnp.float32),
        scratch_types=(
            [pltpu.VMEM((per_w,), jnp.int32),
             pltpu.VMEM((per_w,), jnp.int32),
             pltpu.VMEM((per_w,), jnp.int32)]
            + [pltpu.VMEM((K, PK), jnp.float32) for _ in range(NBUF)]
            + [pltpu.VMEM((K // 8, PK), jnp.float32) for _ in range(NBUF)]
            + [pltpu.SemaphoreType.DMA for _ in range(2 * NBUF)]
        ),
    )
    def k(tab_hbm, idx_hbm, out_hbm, *scr):
        idx_v, idxu_v, offs_v = scr[0], scr[1], scr[2]
        bufs = scr[3:3 + NBUF]
        obufs = scr[3 + NBUF:3 + 2 * NBUF]
        gsems = scr[3 + 2 * NBUF:3 + 3 * NBUF]
        wsems = scr[3 + 3 * NBUF:3 + 4 * NBUF]
        wid = lax.axis_index("s") * NC + lax.axis_index("c")
        base = pl.multiple_of(wid * per_w, 128)
        obase = pl.multiple_of(wid * (per_w // 8), 16)
        pltpu.sync_copy(idx_hbm.at[pl.ds(base, per_w)], idx_v)

        def prep(i, carry):
            v = idx_v[pl.ds(i * LN, LN)]
            idxu_v[pl.ds(i * LN, LN)] = lax.shift_right_logical(v, 3)
            offs_v[pl.ds(i * LN, LN)] = (v & 7) * C
            return carry
        lax.fori_loop(0, per_w // LN, prep, 0, unroll=False)

        def group(g, carry):
            off = g * (NBUF * K)
            gathers = []
            for b in range(NBUF):
                cp = pltpu.make_async_copy(
                    tab_hbm.at[idxu_v.at[pl.ds(off + b * K, K)]],
                    bufs[b], gsems[b])
                cp.start()
                gathers.append(cp)
            writes = []
            for b in range(NBUF):
                gathers[b].wait()
                for q in range(K // LN):
                    tk = q * LN
                    sv = offs_v[pl.ds(off + b * K + tk, LN)]
                    for i in range(LN):
                        kk = tk + i
                        vals = bufs[b][kk, pl.ds(sv[i], C)]
                        obufs[b][kk >> 3, pl.ds((kk & 7) * C, C)] = vals
                wp = pltpu.make_async_copy(
                    obufs[b],
                    out_hbm.at[pl.ds(
                        pl.multiple_of(obase + (off + b * K) // 8, 16),
                        K // 8)],
                    wsems[b])
                wp.start()
                writes.append(wp)
            for b in range(NBUF):
                writes[b].wait()
            return carry

        lax.fori_loop(0, n_groups, group, 0, unroll=False)

    return k(ptable128, idx)


def _gmax(x, lanemod0, bd1, HI):
    r = x
    for s in (1, 2, 4, 8):
        r = jnp.maximum(r, pltpu.roll(r, 128 - s, 1))
    rm = jnp.where(lanemod0, r, 0.0)
    return lax.dot_general(rm, bd1, (((1,), (0,)), ((), ())), precision=HI)


def _crf_body(L, C, em_ref, lab_ref, trans_ref, start_ref, end_ref,
              loss_ref, alpha, gold, ohprev, bd1_s, bdexp_s, bdraw_s, se_s):
    t = pl.program_id(0)
    HI = jax.lax.Precision.HIGHEST
    E = em_ref[0]
    lab = lab_ref[0]
    i1 = lax.broadcasted_iota(jnp.int32, (128, 128), 1)
    lanemod0 = (i1 & (C - 1)) == 0
    oh = (lab == (i1 & (C - 1))).astype(jnp.float32)

    @pl.when(t == 0)
    def _():
        i0 = lax.broadcasted_iota(jnp.int32, (128, 128), 0)
        bd1 = (lax.shift_right_logical(i0, 4)
               == lax.shift_right_logical(i1, 4)).astype(jnp.float32)
        A = ((lax.broadcasted_iota(jnp.int32, (128, C), 0) & (C - 1))
             == lax.broadcasted_iota(jnp.int32, (128, C), 1)
             ).astype(jnp.float32)
        B16 = (lax.broadcasted_iota(jnp.int32, (C, 128), 0)
               == (lax.broadcasted_iota(jnp.int32, (C, 128), 1) & (C - 1))
               ).astype(jnp.float32)
        tt = lax.dot_general(
            lax.dot_general(A, trans_ref[...], (((1,), (0,)), ((), ())),
                            precision=HI),
            B16, (((1,), (0,)), ((), ())), precision=HI)
        bd1_s[...] = bd1
        bdexp_s[...] = jnp.exp(tt) * bd1
        bdraw_s[...] = tt * bd1
        strow = lax.dot_general(start_ref[...], B16, (((0,), (0,)), ((), ())),
                                precision=HI)
        endrow = lax.dot_general(end_ref[...], B16, (((0,), (0,)), ((), ())),
                                 precision=HI)
        se_s[0:1, :] = strow
        se_s[1:2, :] = endrow
        alpha[...] = strow + E
        gold[...] = oh * (strow + E)
        ohprev[...] = oh
        loss_ref[...] = jnp.zeros((1, 1), jnp.float32)

    DEF = jax.lax.Precision.DEFAULT

    @pl.when(t > 0)
    def _():
        bd1 = bd1_s[...]
        a = alpha[...]
        M = _gmax(a, lanemod0, bd1, DEF)
        ea = jnp.exp(a - M)
        S = lax.dot_general(ea, bdexp_s[...], (((1,), (0,)), ((), ())),
                            precision=DEF)
        alpha[...] = M + jnp.log(S) + E
        gtr = lax.dot_general(ohprev[...], bdraw_s[...],
                              (((1,), (0,)), ((), ())), precision=DEF)
        gold[...] = gold[...] + oh * (E + gtr)
        ohprev[...] = oh

    @pl.when(t == L - 1)
    def _():
        bd1 = bd1_s[...]
        endrow = se_s[1:2, :]
        a2 = alpha[...] + endrow
        M2 = _gmax(a2, lanemod0, bd1, HI)
        S2 = lax.dot_general(jnp.exp(a2 - M2), bd1, (((1,), (0,)), ((), ())),
                             precision=HI)
        logZ = jnp.where(lanemod0, M2 + jnp.log(S2), 0.0)
        goldtot = gold[...] + oh * endrow
        g1 = jnp.sum(goldtot - logZ, axis=0, keepdims=True)
        val = jnp.sum(g1, axis=1, keepdims=True)
        loss_ref[...] = -val


def _crf_call(em_p3, lab_p3, trans, start2, end2, interpret=False):
    L = em_p3.shape[0]
    C = trans.shape[0]
    body = functools.partial(_crf_body, L, C)
    loss = pl.pallas_call(
        body,
        grid=(L,),
        in_specs=[
            pl.BlockSpec((1, 128, 128), lambda l: (l, 0, 0)),
            pl.BlockSpec((1, 128, 128), lambda l: (l, 0, 0)),
            pl.BlockSpec((C, C), lambda l: (0, 0)),
            pl.BlockSpec((C, 1), lambda l: (0, 0)),
            pl.BlockSpec((C, 1), lambda l: (0, 0)),
        ],
        out_specs=pl.BlockSpec((1, 1), lambda l: (0, 0)),
        out_shape=jax.ShapeDtypeStruct((1, 1), jnp.float32),
        scratch_shapes=[
            pltpu.VMEM((128, 128), jnp.float32),
            pltpu.VMEM((128, 128), jnp.float32),
            pltpu.VMEM((128, 128), jnp.float32),
            pltpu.VMEM((128, 128), jnp.float32),
            pltpu.VMEM((128, 128), jnp.float32),
            pltpu.VMEM((128, 128), jnp.float32),
            pltpu.VMEM((2, 128), jnp.float32),
        ],
        compiler_params=pltpu.CompilerParams(
            dimension_semantics=("arbitrary",)),
        interpret=interpret,
    )(em_p3, lab_p3, trans, start2, end2)
    return loss


def kernel(x, labels, table, W_fc, b_fc, start_t, end_t, trans):
    B, L = x.shape
    V, D = table.shape
    C = W_fc.shape[1]
    tableT = jnp.swapaxes(table, 0, 1)
    ptableT = _project_table(tableT, jnp.swapaxes(W_fc, 0, 1),
                             b_fc.reshape(C, 1))
    ptable128 = lax.reshape(ptableT, (V // 8, 8 * C), dimensions=(1, 0))
    xT = jnp.swapaxes(x, 0, 1).reshape(-1)
    em_p = _sc_gather(ptable128, xT, C)
    em_p3 = em_p.reshape(L, B // 8, 8 * C)
    labels_T = jnp.swapaxes(labels, 0, 1)
    lab_p3 = jnp.repeat(labels_T.reshape(L, B // 8, 8), C, axis=2)
    loss = _crf_call(em_p3, lab_p3, trans,
                     start_t.reshape(C, 1), end_t.reshape(C, 1))
    logits = jnp.swapaxes(em_p.reshape(L, B, C), 0, 1)
    return (logits, loss[0, 0])


# speedup vs baseline: 1.0661x; 1.0443x over previous
import functools

import jax
import jax.numpy as jnp
from jax import lax
from jax.experimental import pallas as pl
from jax.experimental.pallas import tpu as pltpu
from jax.experimental.pallas import tpu_sc as plsc


def _proj_body(WT_ref, bcol_ref, T_ref, out_ref):
    out_ref[...] = lax.dot_general(
        WT_ref[...], T_ref[...], (((1,), (0,)), ((), ())),
        precision=jax.lax.Precision.HIGHEST) + bcol_ref[...]


def _proj_body_fast(WT_ref, bcol_ref, T_ref, out_ref):
    out_ref[...] = lax.dot_general(
        WT_ref[...], T_ref[...], (((1,), (0,)), ((), ())),
        precision=jax.lax.Precision.DEFAULT) + bcol_ref[...]


def _project_table(tableT, WT, bcol):
    D, V = tableT.shape
    C = WT.shape[0]
    CHUNK = 8192
    grid = (V + CHUNK - 1) // CHUNK
    return pl.pallas_call(
        _proj_body_fast,
        grid=(grid,),
        in_specs=[
            pl.BlockSpec((C, D), lambda i: (0, 0)),
            pl.BlockSpec((C, 1), lambda i: (0, 0)),
            pl.BlockSpec((D, CHUNK), lambda i: (0, i)),
        ],
        out_specs=pl.BlockSpec((C, CHUNK), lambda i: (0, i)),
        out_shape=jax.ShapeDtypeStruct((C, V), jnp.float32),
        compiler_params=pltpu.CompilerParams(
            dimension_semantics=("arbitrary",)),
    )(WT, bcol, tableT)


def _sc_gather(ptable128, idx, C):
    N = idx.shape[0]
    info = plsc.get_sparse_core_info()
    NC, NS, LN = info.num_cores, info.num_subcores, info.num_lanes
    NW = NC * NS
    assert N % (NW * 8) == 0
    per_w = N // NW
    K = 128
    assert per_w % K == 0
    n_chunks = per_w // K
    NBUF = 5 if n_chunks % 5 == 0 else (4 if n_chunks % 4 == 0 else 2)
    assert n_chunks % NBUF == 0
    n_groups = n_chunks // NBUF
    PK = C * 8

    mesh = plsc.VectorSubcoreMesh(core_axis_name="c", subcore_axis_name="s")

    @functools.partial(
        pl.kernel,
        mesh=mesh,
        compiler_params=pltpu.CompilerParams(use_tc_tiling_on_sc=False),
        out_type=jax.ShapeDtypeStruct((N // 8, PK), jnp.float32),
        scratch_types=(
            [pltpu.VMEM((per_w,), jnp.int32),
             pltpu.VMEM((per_w,), jnp.int32),
             pltpu.VMEM((per_w,), jnp.int32)]
            + [pltpu.VMEM((K, PK), jnp.float32) for _ in range(NBUF)]
            + [pltpu.VMEM((K // 8, PK), jnp.float32) for _ in range(NBUF)]
            + [pltpu.SemaphoreType.DMA for _ in range(2 * NBUF)]
        ),
    )
    def k(tab_hbm, idx_hbm, out_hbm, *scr):
        idx_v, idxu_v, offs_v = scr[0], scr[1], scr[2]
        bufs = scr[3:3 + NBUF]
        obufs = scr[3 + NBUF:3 + 2 * NBUF]
        gsems = scr[3 + 2 * NBUF:3 + 3 * NBUF]
        wsems = scr[3 + 3 * NBUF:3 + 4 * NBUF]
        wid = lax.axis_index("s") * NC + lax.axis_index("c")
        base = pl.multiple_of(wid * per_w, 128)
        obase = pl.multiple_of(wid * (per_w // 8), 16)
        pltpu.sync_copy(idx_hbm.at[pl.ds(base, per_w)], idx_v)

        def prep(i, carry):
            v = idx_v[pl.ds(i * LN, LN)]
            idxu_v[pl.ds(i * LN, LN)] = lax.shift_right_logical(v, 3)
            offs_v[pl.ds(i * LN, LN)] = (v & 7) * C
            return carry
        lax.fori_loop(0, per_w // LN, prep, 0, unroll=False)

        def group(g, carry):
            off = g * (NBUF * K)
            gathers = []
            for b in range(NBUF):
                cp = pltpu.make_async_copy(
                    tab_hbm.at[idxu_v.at[pl.ds(off + b * K, K)]],
                    bufs[b], gsems[b])
                cp.start()
                gathers.append(cp)
            writes = []
            for b in range(NBUF):
                gathers[b].wait()
                for q in range(K // LN):
                    tk = q * LN
                    sv = offs_v[pl.ds(off + b * K + tk, LN)]
                    for i in range(LN):
                        kk = tk + i
                        vals = bufs[b][kk, pl.ds(sv[i], C)]
                        obufs[b][kk >> 3, pl.ds((kk & 7) * C, C)] = vals
                wp = pltpu.make_async_copy(
                    obufs[b],
                    out_hbm.at[pl.ds(
                        pl.multiple_of(obase + (off + b * K) // 8, 16),
                        K // 8)],
                    wsems[b])
                wp.start()
                writes.append(wp)
            for b in range(NBUF):
                writes[b].wait()
            return carry

        lax.fori_loop(0, n_groups, group, 0, unroll=False)

    return k(ptable128, idx)


def _gmax(x, lanemod0, bd1, HI):
    r = x
    for s in (1, 2, 4, 8):
        r = jnp.maximum(r, pltpu.roll(r, 128 - s, 1))
    rm = jnp.where(lanemod0, r, 0.0)
    return lax.dot_general(rm, bd1, (((1,), (0,)), ((), ())), precision=HI)


def _crf_body(L, C, em_ref, lab_ref, trans_ref, start_ref, end_ref,
              loss_ref, alpha, gold, ohprev, bd1_s, bdexp_s, bdraw_s, se_s):
    t = pl.program_id(0)
    HI = jax.lax.Precision.HIGHEST
    E = em_ref[0]
    lab = lab_ref[0]
    i1 = lax.broadcasted_iota(jnp.int32, (128, 128), 1)
    lanemod0 = (i1 & (C - 1)) == 0
    oh = (lab == (i1 & (C - 1))).astype(jnp.float32)

    @pl.when(t == 0)
    def _():
        i0 = lax.broadcasted_iota(jnp.int32, (128, 128), 0)
        bd1 = (lax.shift_right_logical(i0, 4)
               == lax.shift_right_logical(i1, 4)).astype(jnp.float32)
        A = ((lax.broadcasted_iota(jnp.int32, (128, C), 0) & (C - 1))
             == lax.broadcasted_iota(jnp.int32, (128, C), 1)
             ).astype(jnp.float32)
        B16 = (lax.broadcasted_iota(jnp.int32, (C, 128), 0)
               == (lax.broadcasted_iota(jnp.int32, (C, 128), 1) & (C - 1))
               ).astype(jnp.float32)
        tt = lax.dot_general(
            lax.dot_general(A, trans_ref[...], (((1,), (0,)), ((), ())),
                            precision=HI),
            B16, (((1,), (0,)), ((), ())), precision=HI)
        bd1_s[...] = bd1
        bdexp_s[...] = jnp.exp(tt) * bd1
        bdraw_s[...] = tt * bd1
        strow = lax.dot_general(start_ref[...], B16, (((0,), (0,)), ((), ())),
                                precision=HI)
        endrow = lax.dot_general(end_ref[...], B16, (((0,), (0,)), ((), ())),
                                 precision=HI)
        se_s[0:1, :] = strow
        se_s[1:2, :] = endrow
        alpha[...] = strow + E
        gold[...] = oh * (strow + E)
        ohprev[...] = oh
        loss_ref[...] = jnp.zeros((1, 1), jnp.float32)

    DEF = jax.lax.Precision.DEFAULT

    @pl.when(t > 0)
    def _():
        bd1 = bd1_s[...]
        a = alpha[...]
        M = _gmax(a, lanemod0, bd1, DEF)
        ea = jnp.exp(a - M)
        S = lax.dot_general(ea, bdexp_s[...], (((1,), (0,)), ((), ())),
                            precision=DEF)
        alpha[...] = M + jnp.log(S) + E
        gtr = lax.dot_general(ohprev[...], bdraw_s[...],
                              (((1,), (0,)), ((), ())), precision=DEF)
        gold[...] = gold[...] + oh * (E + gtr)
        ohprev[...] = oh

    @pl.when(t == L - 1)
    def _():
        bd1 = bd1_s[...]
        endrow = se_s[1:2, :]
        a2 = alpha[...] + endrow
        M2 = _gmax(a2, lanemod0, bd1, HI)
        S2 = lax.dot_general(jnp.exp(a2 - M2), bd1, (((1,), (0,)), ((), ())),
                             precision=HI)
        logZ = jnp.where(lanemod0, M2 + jnp.log(S2), 0.0)
        goldtot = gold[...] + oh * endrow
        g1 = jnp.sum(goldtot - logZ, axis=0, keepdims=True)
        val = jnp.sum(g1, axis=1, keepdims=True)
        loss_ref[...] = -val


def _crf_call(em_p3, lab_p3, trans, start2, end2, interpret=False):
    L = em_p3.shape[0]
    C = trans.shape[0]
    body = functools.partial(_crf_body, L, C)
    loss = pl.pallas_call(
        body,
        grid=(L,),
        in_specs=[
            pl.BlockSpec((1, 128, 128), lambda l: (l, 0, 0)),
            pl.BlockSpec((1, 128, 128), lambda l: (l, 0, 0)),
            pl.BlockSpec((C, C), lambda l: (0, 0)),
            pl.BlockSpec((C, 1), lambda l: (0, 0)),
            pl.BlockSpec((C, 1), lambda l: (0, 0)),
        ],
        out_specs=pl.BlockSpec((1, 1), lambda l: (0, 0)),
        out_shape=jax.ShapeDtypeStruct((1, 1), jnp.float32),
        scratch_shapes=[
            pltpu.VMEM((128, 128), jnp.float32),
            pltpu.VMEM((128, 128), jnp.float32),
            pltpu.VMEM((128, 128), jnp.float32),
            pltpu.VMEM((128, 128), jnp.float32),
            pltpu.VMEM((128, 128), jnp.float32),
            pltpu.VMEM((128, 128), jnp.float32),
            pltpu.VMEM((2, 128), jnp.float32),
        ],
        compiler_params=pltpu.CompilerParams(
            dimension_semantics=("arbitrary",)),
        interpret=interpret,
    )(em_p3, lab_p3, trans, start2, end2)
    return loss


def kernel(x, labels, table, W_fc, b_fc, start_t, end_t, trans):
    B, L = x.shape
    V, D = table.shape
    C = W_fc.shape[1]
    tableT = jnp.swapaxes(table, 0, 1)
    ptableT = _project_table(tableT, jnp.swapaxes(W_fc, 0, 1),
                             b_fc.reshape(C, 1))
    ptable128 = lax.reshape(ptableT, (V // 8, 8 * C), dimensions=(1, 0))
    xT = jnp.swapaxes(x, 0, 1).reshape(-1)
    em_p = _sc_gather(ptable128, xT, C)
    em_p3 = em_p.reshape(L, B // 8, 8 * C)
    labels_T = jnp.swapaxes(labels, 0, 1)
    lab_p3 = jnp.repeat(labels_T.reshape(L, B // 8, 8), C, axis=2)
    loss = _crf_call(em_p3, lab_p3, trans,
                     start_t.reshape(C, 1), end_t.reshape(C, 1))
    logits = jnp.swapaxes(em_p.reshape(L, B, C), 0, 1)
    return (logits, loss[0, 0])


# speedup vs baseline: 1.6572x; 1.5545x over previous
import functools

import jax
import jax.numpy as jnp
from jax import lax
from jax.experimental import pallas as pl
from jax.experimental.pallas import tpu as pltpu
from jax.experimental.pallas import tpu_sc as plsc


def _proj_body(WT_ref, bcol_ref, T_ref, out_ref):
    out_ref[...] = lax.dot_general(
        WT_ref[...], T_ref[...], (((1,), (0,)), ((), ())),
        precision=jax.lax.Precision.HIGHEST) + bcol_ref[...]


def _proj_body_fast(WT_ref, bcol_ref, T_ref, out_ref):
    out_ref[...] = lax.dot_general(
        WT_ref[...], T_ref[...], (((1,), (0,)), ((), ())),
        precision=jax.lax.Precision.DEFAULT) + bcol_ref[...]


def _projY_body(Wpad_ref, bpad_ref, T_ref, out_ref):
    out_ref[...] = lax.dot_general(
        T_ref[...], Wpad_ref[...], (((0,), (0,)), ((), ())),
        precision=jax.lax.Precision.DEFAULT) + bpad_ref[...]


def _project_tableY(tableT, Wpad, bpad):
    D, V = tableT.shape
    CHUNK = 8192
    grid = (V + CHUNK - 1) // CHUNK
    return pl.pallas_call(
        _projY_body,
        grid=(grid,),
        in_specs=[
            pl.BlockSpec((D, 128), lambda i: (0, 0)),
            pl.BlockSpec((1, 128), lambda i: (0, 0)),
            pl.BlockSpec((D, CHUNK), lambda i: (0, i)),
        ],
        out_specs=pl.BlockSpec((CHUNK, 128), lambda i: (i, 0)),
        out_shape=jax.ShapeDtypeStruct((grid * CHUNK, 128), jnp.float32),
        compiler_params=pltpu.CompilerParams(
            dimension_semantics=("arbitrary",)),
    )(Wpad, bpad, tableT)


def _project_table(tableT, WT, bcol):
    D, V = tableT.shape
    C = WT.shape[0]
    CHUNK = 8192
    grid = (V + CHUNK - 1) // CHUNK
    return pl.pallas_call(
        _proj_body_fast,
        grid=(grid,),
        in_specs=[
            pl.BlockSpec((C, D), lambda i: (0, 0)),
            pl.BlockSpec((C, 1), lambda i: (0, 0)),
            pl.BlockSpec((D, CHUNK), lambda i: (0, i)),
        ],
        out_specs=pl.BlockSpec((C, CHUNK), lambda i: (0, i)),
        out_shape=jax.ShapeDtypeStruct((C, V), jnp.float32),
        compiler_params=pltpu.CompilerParams(
            dimension_semantics=("arbitrary",)),
    )(WT, bcol, tableT)


def _sc_gather(ptable128, idx, C):
    N = idx.shape[0]
    info = plsc.get_sparse_core_info()
    NC, NS, LN = info.num_cores, info.num_subcores, info.num_lanes
    NW = NC * NS
    assert N % (NW * 8) == 0
    per_w = N // NW
    K = 128
    assert per_w % K == 0
    n_chunks = per_w // K
    NBUF = 5 if n_chunks % 5 == 0 else (4 if n_chunks % 4 == 0 else 2)
    assert n_chunks % NBUF == 0
    n_groups = n_chunks // NBUF
    PK = C * 8

    mesh = plsc.VectorSubcoreMesh(core_axis_name="c", subcore_axis_name="s")

    @functools.partial(
        pl.kernel,
        mesh=mesh,
        compiler_params=pltpu.CompilerParams(use_tc_tiling_on_sc=False),
        out_type=jax.ShapeDtypeStruct((N // 8, PK), jnp.float32),
        scratch_types=(
            [pltpu.VMEM((per_w,), jnp.int32)]
            + [pltpu.VMEM((K, PK), jnp.float32) for _ in range(NBUF)]
            + [pltpu.VMEM((K // 8, PK), jnp.float32) for _ in range(NBUF)]
            + [pltpu.SemaphoreType.DMA for _ in range(2 * NBUF)]
        ),
    )
    def k(tab_hbm, idx_hbm, out_hbm, *scr):
        idx_v = scr[0]
        bufs = scr[1:1 + NBUF]
        obufs = scr[1 + NBUF:1 + 2 * NBUF]
        gsems = scr[1 + 2 * NBUF:1 + 3 * NBUF]
        wsems = scr[1 + 3 * NBUF:1 + 4 * NBUF]
        wid = lax.axis_index("s") * NC + lax.axis_index("c")
        base = pl.multiple_of(wid * per_w, 128)
        obase = pl.multiple_of(wid * (per_w // 8), 16)
        pltpu.sync_copy(idx_hbm.at[pl.ds(base, per_w)], idx_v)

        def group(g, carry):
            off = g * (NBUF * K)
            gathers = []
            for b in range(NBUF):
                cp = pltpu.make_async_copy(
                    tab_hbm.at[idx_v.at[pl.ds(off + b * K, K)]],
                    bufs[b], gsems[b])
                cp.start()
                gathers.append(cp)
            writes = []
            for b in range(NBUF):
                gathers[b].wait()
                for kk in range(K):
                    obufs[b][kk >> 3, pl.ds((kk & 7) * C, C)] = (
                        bufs[b][kk, pl.ds(0, C)])
                wp = pltpu.make_async_copy(
                    obufs[b],
                    out_hbm.at[pl.ds(
                        pl.multiple_of(obase + (off + b * K) // 8, 16),
                        K // 8)],
                    wsems[b])
                wp.start()
                writes.append(wp)
            for b in range(NBUF):
                writes[b].wait()
            return carry

        lax.fori_loop(0, n_groups, group, 0, unroll=False)

    return k(ptable128, idx)


def _gmax(x, lanemod0, bd1, HI):
    r = x
    for s in (1, 2, 4, 8):
        r = jnp.maximum(r, pltpu.roll(r, 128 - s, 1))
    rm = jnp.where(lanemod0, r, 0.0)
    return lax.dot_general(rm, bd1, (((1,), (0,)), ((), ())), precision=HI)


def _crf_body(L, C, em_ref, lab_ref, trans_ref, start_ref, end_ref,
              loss_ref, alpha, gold, ohprev, bd1_s, bdexp_s, bdraw_s, se_s):
    t = pl.program_id(0)
    HI = jax.lax.Precision.HIGHEST
    E = em_ref[0]
    lab = lab_ref[0]
    i1 = lax.broadcasted_iota(jnp.int32, (128, 128), 1)
    lanemod0 = (i1 & (C - 1)) == 0
    oh = (lab == (i1 & (C - 1))).astype(jnp.float32)

    @pl.when(t == 0)
    def _():
        i0 = lax.broadcasted_iota(jnp.int32, (128, 128), 0)
        bd1 = (lax.shift_right_logical(i0, 4)
               == lax.shift_right_logical(i1, 4)).astype(jnp.float32)
        A = ((lax.broadcasted_iota(jnp.int32, (128, C), 0) & (C - 1))
             == lax.broadcasted_iota(jnp.int32, (128, C), 1)
             ).astype(jnp.float32)
        B16 = (lax.broadcasted_iota(jnp.int32, (C, 128), 0)
               == (lax.broadcasted_iota(jnp.int32, (C, 128), 1) & (C - 1))
               ).astype(jnp.float32)
        tt = lax.dot_general(
            lax.dot_general(A, trans_ref[...], (((1,), (0,)), ((), ())),
                            precision=HI),
            B16, (((1,), (0,)), ((), ())), precision=HI)
        bd1_s[...] = bd1
        bdexp_s[...] = jnp.exp(tt) * bd1
        bdraw_s[...] = tt * bd1
        strow = lax.dot_general(start_ref[...], B16, (((0,), (0,)), ((), ())),
                                precision=HI)
        endrow = lax.dot_general(end_ref[...], B16, (((0,), (0,)), ((), ())),
                                 precision=HI)
        se_s[0:1, :] = strow
        se_s[1:2, :] = endrow
        alpha[...] = strow + E
        gold[...] = oh * (strow + E)
        ohprev[...] = oh
        loss_ref[...] = jnp.zeros((1, 1), jnp.float32)

    DEF = jax.lax.Precision.DEFAULT

    @pl.when(t > 0)
    def _():
        bd1 = bd1_s[...]
        a = alpha[...]
        M = _gmax(a, lanemod0, bd1, DEF)
        ea = jnp.exp(a - M)
        S = lax.dot_general(ea, bdexp_s[...], (((1,), (0,)), ((), ())),
                            precision=DEF)
        alpha[...] = M + jnp.log(S) + E
        gtr = lax.dot_general(ohprev[...], bdraw_s[...],
                              (((1,), (0,)), ((), ())), precision=DEF)
        gold[...] = gold[...] + oh * (E + gtr)
        ohprev[...] = oh

    @pl.when(t == L - 1)
    def _():
        bd1 = bd1_s[...]
        endrow = se_s[1:2, :]
        a2 = alpha[...] + endrow
        M2 = _gmax(a2, lanemod0, bd1, HI)
        S2 = lax.dot_general(jnp.exp(a2 - M2), bd1, (((1,), (0,)), ((), ())),
                             precision=HI)
        logZ = jnp.where(lanemod0, M2 + jnp.log(S2), 0.0)
        goldtot = gold[...] + oh * endrow
        g1 = jnp.sum(goldtot - logZ, axis=0, keepdims=True)
        val = jnp.sum(g1, axis=1, keepdims=True)
        loss_ref[...] = -val


def _crf_call(em_p3, lab_p3, trans, start2, end2, interpret=False):
    L = em_p3.shape[0]
    C = trans.shape[0]
    body = functools.partial(_crf_body, L, C)
    loss = pl.pallas_call(
        body,
        grid=(L,),
        in_specs=[
            pl.BlockSpec((1, 128, 128), lambda l: (l, 0, 0)),
            pl.BlockSpec((1, 128, 128), lambda l: (l, 0, 0)),
            pl.BlockSpec((C, C), lambda l: (0, 0)),
            pl.BlockSpec((C, 1), lambda l: (0, 0)),
            pl.BlockSpec((C, 1), lambda l: (0, 0)),
        ],
        out_specs=pl.BlockSpec((1, 1), lambda l: (0, 0)),
        out_shape=jax.ShapeDtypeStruct((1, 1), jnp.float32),
        scratch_shapes=[
            pltpu.VMEM((128, 128), jnp.float32),
            pltpu.VMEM((128, 128), jnp.float32),
            pltpu.VMEM((128, 128), jnp.float32),
            pltpu.VMEM((128, 128), jnp.float32),
            pltpu.VMEM((128, 128), jnp.float32),
            pltpu.VMEM((128, 128), jnp.float32),
            pltpu.VMEM((2, 128), jnp.float32),
        ],
        compiler_params=pltpu.CompilerParams(
            dimension_semantics=("arbitrary",)),
        interpret=interpret,
    )(em_p3, lab_p3, trans, start2, end2)
    return loss


def kernel(x, labels, table, W_fc, b_fc, start_t, end_t, trans):
    B, L = x.shape
    V, D = table.shape
    C = W_fc.shape[1]
    tableT = jnp.swapaxes(table, 0, 1)
    Wpad = jnp.pad(W_fc, ((0, 0), (0, 128 - C)))
    bpad = jnp.pad(b_fc, (0, 128 - C)).reshape(1, 128)
    ptable128 = _project_tableY(tableT, Wpad, bpad)
    xT = jnp.swapaxes(x, 0, 1).reshape(-1)
    em_p = _sc_gather(ptable128, xT, C)
    em_p3 = em_p.reshape(L, B // 8, 8 * C)
    labels_T = jnp.swapaxes(labels, 0, 1)
    lab_p3 = jnp.repeat(labels_T.reshape(L, B // 8, 8), C, axis=2)
    loss = _crf_call(em_p3, lab_p3, trans,
                     start_t.reshape(C, 1), end_t.reshape(C, 1))
    logits = jnp.swapaxes(em_p.reshape(L, B, C), 0, 1)
    return (logits, loss[0, 0])


# speedup vs baseline: 1.6604x; 1.0019x over previous
import functools

import jax
import jax.numpy as jnp
from jax import lax
from jax.experimental import pallas as pl
from jax.experimental.pallas import tpu as pltpu
from jax.experimental.pallas import tpu_sc as plsc


def _projY_body(Wpad_ref, bpad_ref, T_ref, out_ref):
    out_ref[...] = lax.dot_general(
        T_ref[...], Wpad_ref[...], (((0,), (0,)), ((), ())),
        precision=jax.lax.Precision.DEFAULT) + bpad_ref[...]


def _project_tableY(tableT, Wpad, bpad):
    D, V = tableT.shape
    CHUNK = 8192
    grid = (V + CHUNK - 1) // CHUNK
    return pl.pallas_call(
        _projY_body,
        grid=(grid,),
        in_specs=[
            pl.BlockSpec((D, 128), lambda i: (0, 0)),
            pl.BlockSpec((1, 128), lambda i: (0, 0)),
            pl.BlockSpec((D, CHUNK), lambda i: (0, i)),
        ],
        out_specs=pl.BlockSpec((CHUNK, 128), lambda i: (i, 0)),
        out_shape=jax.ShapeDtypeStruct((grid * CHUNK, 128), jnp.float32),
        compiler_params=pltpu.CompilerParams(
            dimension_semantics=("arbitrary",)),
    )(Wpad, bpad, tableT)


def _sc_gather(ptable128, idx, C):
    N = idx.shape[0]
    info = plsc.get_sparse_core_info()
    NC, NS, LN = info.num_cores, info.num_subcores, info.num_lanes
    NW = NC * NS
    assert N % (NW * 8) == 0
    per_w = N // NW
    K = 128
    assert per_w % K == 0
    n_chunks = per_w // K
    NBUF = 5 if n_chunks % 5 == 0 else (4 if n_chunks % 4 == 0 else 2)
    assert n_chunks % NBUF == 0
    n_groups = n_chunks // NBUF
    PK = C * 8

    mesh = plsc.VectorSubcoreMesh(core_axis_name="c", subcore_axis_name="s")

    @functools.partial(
        pl.kernel,
        mesh=mesh,
        compiler_params=pltpu.CompilerParams(use_tc_tiling_on_sc=False),
        out_type=jax.ShapeDtypeStruct((N // 8, PK), jnp.float32),
        scratch_types=(
            [pltpu.VMEM((per_w,), jnp.int32)]
            + [pltpu.VMEM((K, PK), jnp.float32) for _ in range(NBUF)]
            + [pltpu.VMEM((K // 8, PK), jnp.float32) for _ in range(NBUF)]
            + [pltpu.SemaphoreType.DMA for _ in range(2 * NBUF)]
        ),
    )
    def k(tab_hbm, idx_hbm, out_hbm, *scr):
        idx_v = scr[0]
        bufs = scr[1:1 + NBUF]
        obufs = scr[1 + NBUF:1 + 2 * NBUF]
        gsems = scr[1 + 2 * NBUF:1 + 3 * NBUF]
        wsems = scr[1 + 3 * NBUF:1 + 4 * NBUF]
        wid = lax.axis_index("s") * NC + lax.axis_index("c")
        base = pl.multiple_of(wid * per_w, 128)
        obase = pl.multiple_of(wid * (per_w // 8), 16)
        pltpu.sync_copy(idx_hbm.at[pl.ds(base, per_w)], idx_v)

        def group(g, carry):
            off = g * (NBUF * K)
            gathers = []
            for b in range(NBUF):
                cp = pltpu.make_async_copy(
                    tab_hbm.at[idx_v.at[pl.ds(off + b * K, K)]],
                    bufs[b], gsems[b])
                cp.start()
                gathers.append(cp)
            writes = []
            for b in range(NBUF):
                gathers[b].wait()
                for kk in range(K):
                    obufs[b][kk >> 3, pl.ds((kk & 7) * C, C)] = (
                        bufs[b][kk, pl.ds(0, C)])
                wp = pltpu.make_async_copy(
                    obufs[b],
                    out_hbm.at[pl.ds(
                        pl.multiple_of(obase + (off + b * K) // 8, 16),
                        K // 8)],
                    wsems[b])
                wp.start()
                writes.append(wp)
            for b in range(NBUF):
                writes[b].wait()
            return carry

        lax.fori_loop(0, n_groups, group, 0, unroll=False)

    return k(ptable128, idx)


def _gmax(x, lanemod0, bd1, HI):
    r = x
    for s in (1, 2, 4, 8):
        r = jnp.maximum(r, pltpu.roll(r, 128 - s, 1))
    rm = jnp.where(lanemod0, r, 0.0)
    return lax.dot_general(rm, bd1, (((1,), (0,)), ((), ())), precision=HI)


def _crf_body(L, C, em_ref, lab_ref, trans_ref, start_ref, end_ref,
              loss_ref, alpha, gold, ohprev, bd1_s, bdexp_s, bdraw_s, se_s):
    t = pl.program_id(0)
    HI = jax.lax.Precision.HIGHEST
    E = em_ref[0]
    lab = lab_ref[0]
    i1 = lax.broadcasted_iota(jnp.int32, (128, 128), 1)
    lanemod0 = (i1 & (C - 1)) == 0
    oh = (lab == (i1 & (C - 1))).astype(jnp.float32)

    @pl.when(t == 0)
    def _():
        i0 = lax.broadcasted_iota(jnp.int32, (128, 128), 0)
        bd1 = (lax.shift_right_logical(i0, 4)
               == lax.shift_right_logical(i1, 4)).astype(jnp.float32)
        A = ((lax.broadcasted_iota(jnp.int32, (128, C), 0) & (C - 1))
             == lax.broadcasted_iota(jnp.int32, (128, C), 1)
             ).astype(jnp.float32)
        B16 = (lax.broadcasted_iota(jnp.int32, (C, 128), 0)
               == (lax.broadcasted_iota(jnp.int32, (C, 128), 1) & (C - 1))
               ).astype(jnp.float32)
        tt = lax.dot_general(
            lax.dot_general(A, trans_ref[...], (((1,), (0,)), ((), ())),
                            precision=HI),
            B16, (((1,), (0,)), ((), ())), precision=HI)
        bd1_s[...] = bd1
        bdexp_s[...] = jnp.exp(tt) * bd1
        bdraw_s[...] = tt * bd1
        strow = lax.dot_general(start_ref[...], B16, (((0,), (0,)), ((), ())),
                                precision=HI)
        endrow = lax.dot_general(end_ref[...], B16, (((0,), (0,)), ((), ())),
                                 precision=HI)
        se_s[0:1, :] = strow
        se_s[1:2, :] = endrow
        alpha[...] = strow + E
        gold[...] = oh * (strow + E)
        ohprev[...] = oh
        loss_ref[...] = jnp.zeros((1, 1), jnp.float32)

    DEF = jax.lax.Precision.DEFAULT

    @pl.when(t > 0)
    def _():
        bd1 = bd1_s[...]
        a = alpha[...]
        M = _gmax(a, lanemod0, bd1, DEF)
        ea = jnp.exp(a - M)
        S = lax.dot_general(ea, bdexp_s[...], (((1,), (0,)), ((), ())),
                            precision=DEF)
        alpha[...] = M + jnp.log(S) + E
        gtr = lax.dot_general(ohprev[...], bdraw_s[...],
                              (((1,), (0,)), ((), ())), precision=DEF)
        gold[...] = gold[...] + oh * (E + gtr)
        ohprev[...] = oh

    @pl.when(t == L - 1)
    def _():
        bd1 = bd1_s[...]
        endrow = se_s[1:2, :]
        a2 = alpha[...] + endrow
        M2 = _gmax(a2, lanemod0, bd1, HI)
        S2 = lax.dot_general(jnp.exp(a2 - M2), bd1, (((1,), (0,)), ((), ())),
                             precision=HI)
        logZ = jnp.where(lanemod0, M2 + jnp.log(S2), 0.0)
        goldtot = gold[...] + oh * endrow
        g1 = jnp.sum(goldtot - logZ, axis=0, keepdims=True)
        val = jnp.sum(g1, axis=1, keepdims=True)
        loss_ref[...] = -val


def _crf_call(em_p3, lab_p3, trans, start2, end2):
    L = em_p3.shape[0]
    C = trans.shape[0]
    body = functools.partial(_crf_body, L, C)
    loss = pl.pallas_call(
        body,
        grid=(L,),
        in_specs=[
            pl.BlockSpec((1, 128, 128), lambda l: (l, 0, 0)),
            pl.BlockSpec((1, 128, 128), lambda l: (l, 0, 0)),
            pl.BlockSpec((C, C), lambda l: (0, 0)),
            pl.BlockSpec((C, 1), lambda l: (0, 0)),
            pl.BlockSpec((C, 1), lambda l: (0, 0)),
        ],
        out_specs=pl.BlockSpec((1, 1), lambda l: (0, 0)),
        out_shape=jax.ShapeDtypeStruct((1, 1), jnp.float32),
        scratch_shapes=[
            pltpu.VMEM((128, 128), jnp.float32),
            pltpu.VMEM((128, 128), jnp.float32),
            pltpu.VMEM((128, 128), jnp.float32),
            pltpu.VMEM((128, 128), jnp.float32),
            pltpu.VMEM((128, 128), jnp.float32),
            pltpu.VMEM((128, 128), jnp.float32),
            pltpu.VMEM((2, 128), jnp.float32),
        ],
        compiler_params=pltpu.CompilerParams(
            dimension_semantics=("arbitrary",)),
    )(em_p3, lab_p3, trans, start2, end2)
    return loss


def kernel(x, labels, table, W_fc, b_fc, start_t, end_t, trans):
    B, L = x.shape
    V, D = table.shape
    C = W_fc.shape[1]
    tableT = jnp.swapaxes(table, 0, 1)
    Wpad = jnp.pad(W_fc, ((0, 0), (0, 128 - C)))
    bpad = jnp.pad(b_fc, (0, 128 - C)).reshape(1, 128)
    ptable128 = _project_tableY(tableT, Wpad, bpad)
    xT = jnp.swapaxes(x, 0, 1).reshape(-1)
    em_p = _sc_gather(ptable128, xT, C)
    em_p3 = em_p.reshape(L, B // 8, 8 * C)
    labels_T = jnp.swapaxes(labels, 0, 1)
    lab_p3 = jnp.repeat(labels_T.reshape(L, B // 8, 8), C, axis=2)
    loss = _crf_call(em_p3, lab_p3, trans,
                     start_t.reshape(C, 1), end_t.reshape(C, 1))
    logits = jnp.swapaxes(em_p.reshape(L, B, C), 0, 1)
    return (logits, loss[0, 0])


# speedup vs baseline: 1.6613x; 1.0005x over previous
import functools

import jax
import jax.numpy as jnp
from jax import lax
from jax.experimental import pallas as pl
from jax.experimental.pallas import tpu as pltpu
from jax.experimental.pallas import tpu_sc as plsc


def _projY_body(Wpad_ref, bpad_ref, T_ref, out_ref):
    out_ref[...] = lax.dot_general(
        T_ref[...], Wpad_ref[...], (((0,), (0,)), ((), ())),
        precision=jax.lax.Precision.DEFAULT) + bpad_ref[...]


def _project_tableY(tableT, Wpad, bpad):
    D, V = tableT.shape
    CHUNK = 8192
    grid = (V + CHUNK - 1) // CHUNK
    return pl.pallas_call(
        _projY_body,
        grid=(grid,),
        in_specs=[
            pl.BlockSpec((D, 128), lambda i: (0, 0)),
            pl.BlockSpec((1, 128), lambda i: (0, 0)),
            pl.BlockSpec((D, CHUNK), lambda i: (0, i)),
        ],
        out_specs=pl.BlockSpec((CHUNK, 128), lambda i: (i, 0)),
        out_shape=jax.ShapeDtypeStruct((grid * CHUNK, 128), jnp.float32),
        compiler_params=pltpu.CompilerParams(
            dimension_semantics=("arbitrary",)),
    )(Wpad, bpad, tableT)


def _sc_gather(ptable128, idx, C):
    N = idx.shape[0]
    info = plsc.get_sparse_core_info()
    NC, NS, LN = info.num_cores, info.num_subcores, info.num_lanes
    NW = NC * NS
    assert N % (NW * 8) == 0
    per_w = N // NW
    K = 128
    assert per_w % K == 0
    n_chunks = per_w // K
    NBUF = 5 if n_chunks % 5 == 0 else (4 if n_chunks % 4 == 0 else 2)
    assert n_chunks % NBUF == 0
    n_groups = n_chunks // NBUF
    PK = C * 8

    mesh = plsc.VectorSubcoreMesh(core_axis_name="c", subcore_axis_name="s")

    @functools.partial(
        pl.kernel,
        mesh=mesh,
        out_type=jax.ShapeDtypeStruct((N // 8, PK), jnp.float32),
        scratch_types=(
            [pltpu.VMEM((per_w,), jnp.int32)]
            + [pltpu.VMEM((K, PK), jnp.float32) for _ in range(NBUF)]
            + [pltpu.VMEM((K // 8, PK), jnp.float32) for _ in range(NBUF)]
            + [pltpu.SemaphoreType.DMA for _ in range(2 * NBUF)]
        ),
    )
    def k(tab_hbm, idx_hbm, out_hbm, *scr):
        idx_v = scr[0]
        bufs = scr[1:1 + NBUF]
        obufs = scr[1 + NBUF:1 + 2 * NBUF]
        gsems = scr[1 + 2 * NBUF:1 + 3 * NBUF]
        wsems = scr[1 + 3 * NBUF:1 + 4 * NBUF]
        wid = lax.axis_index("s") * NC + lax.axis_index("c")
        base = pl.multiple_of(wid * per_w, 128)
        obase = pl.multiple_of(wid * (per_w // 8), 16)
        pltpu.sync_copy(idx_hbm.at[pl.ds(base, per_w)], idx_v)

        def group(g, carry):
            off = g * (NBUF * K)
            gathers = []
            for b in range(NBUF):
                cp = pltpu.make_async_copy(
                    tab_hbm.at[idx_v.at[pl.ds(off + b * K, K)]],
                    bufs[b], gsems[b])
                cp.start()
                gathers.append(cp)
            writes = []
            for b in range(NBUF):
                gathers[b].wait()
                for kk in range(K):
                    obufs[b][kk >> 3, pl.ds((kk & 7) * C, C)] = (
                        bufs[b][kk, pl.ds(0, C)])
                wp = pltpu.make_async_copy(
                    obufs[b],
                    out_hbm.at[pl.ds(
                        pl.multiple_of(obase + (off + b * K) // 8, 16),
                        K // 8)],
                    wsems[b])
                wp.start()
                writes.append(wp)
            for b in range(NBUF):
                writes[b].wait()
            return carry

        lax.fori_loop(0, n_groups, group, 0, unroll=False)

    return k(ptable128, idx)


def _gmax(x, lanemod0, bd1, HI):
    r = x
    for s in (1, 2, 4, 8):
        r = jnp.maximum(r, pltpu.roll(r, 128 - s, 1))
    rm = jnp.where(lanemod0, r, 0.0)
    return lax.dot_general(rm, bd1, (((1,), (0,)), ((), ())), precision=HI)


def _crf_body(L, C, em_ref, lab_ref, trans_ref, start_ref, end_ref,
              loss_ref, alpha, gold, ohprev, bd1_s, bdexp_s, bdraw_s, se_s):
    t = pl.program_id(0)
    HI = jax.lax.Precision.HIGHEST
    E = em_ref[0]
    lab = lab_ref[0]
    i1 = lax.broadcasted_iota(jnp.int32, (128, 128), 1)
    lanemod0 = (i1 & (C - 1)) == 0
    oh = (lab == (i1 & (C - 1))).astype(jnp.float32)

    @pl.when(t == 0)
    def _():
        i0 = lax.broadcasted_iota(jnp.int32, (128, 128), 0)
        bd1 = (lax.shift_right_logical(i0, 4)
               == lax.shift_right_logical(i1, 4)).astype(jnp.float32)
        A = ((lax.broadcasted_iota(jnp.int32, (128, C), 0) & (C - 1))
             == lax.broadcasted_iota(jnp.int32, (128, C), 1)
             ).astype(jnp.float32)
        B16 = (lax.broadcasted_iota(jnp.int32, (C, 128), 0)
               == (lax.broadcasted_iota(jnp.int32, (C, 128), 1) & (C - 1))
               ).astype(jnp.float32)
        tt = lax.dot_general(
            lax.dot_general(A, trans_ref[...], (((1,), (0,)), ((), ())),
                            precision=HI),
            B16, (((1,), (0,)), ((), ())), precision=HI)
        bd1_s[...] = bd1
        bdexp_s[...] = jnp.exp(tt) * bd1
        bdraw_s[...] = tt * bd1
        strow = lax.dot_general(start_ref[...], B16, (((0,), (0,)), ((), ())),
                                precision=HI)
        endrow = lax.dot_general(end_ref[...], B16, (((0,), (0,)), ((), ())),
                                 precision=HI)
        se_s[0:1, :] = strow
        se_s[1:2, :] = endrow
        alpha[...] = strow + E
        gold[...] = oh * (strow + E)
        ohprev[...] = oh
        loss_ref[...] = jnp.zeros((1, 1), jnp.float32)

    DEF = jax.lax.Precision.DEFAULT

    @pl.when(t > 0)
    def _():
        bd1 = bd1_s[...]
        a = alpha[...]
        M = _gmax(a, lanemod0, bd1, DEF)
        ea = jnp.exp(a - M)
        S = lax.dot_general(ea, bdexp_s[...], (((1,), (0,)), ((), ())),
                            precision=DEF)
        alpha[...] = M + jnp.log(S) + E
        gtr = lax.dot_general(ohprev[...], bdraw_s[...],
                              (((1,), (0,)), ((), ())), precision=DEF)
        gold[...] = gold[...] + oh * (E + gtr)
        ohprev[...] = oh

    @pl.when(t == L - 1)
    def _():
        bd1 = bd1_s[...]
        endrow = se_s[1:2, :]
        a2 = alpha[...] + endrow
        M2 = _gmax(a2, lanemod0, bd1, HI)
        S2 = lax.dot_general(jnp.exp(a2 - M2), bd1, (((1,), (0,)), ((), ())),
                             precision=HI)
        logZ = jnp.where(lanemod0, M2 + jnp.log(S2), 0.0)
        goldtot = gold[...] + oh * endrow
        g1 = jnp.sum(goldtot - logZ, axis=0, keepdims=True)
        val = jnp.sum(g1, axis=1, keepdims=True)
        loss_ref[...] = -val


def _crf_call(em_p3, lab_p3, trans, start2, end2):
    L = em_p3.shape[0]
    C = trans.shape[0]
    body = functools.partial(_crf_body, L, C)
    loss = pl.pallas_call(
        body,
        grid=(L,),
        in_specs=[
            pl.BlockSpec((1, 128, 128), lambda l: (l, 0, 0)),
            pl.BlockSpec((1, 128, 128), lambda l: (l, 0, 0)),
            pl.BlockSpec((C, C), lambda l: (0, 0)),
            pl.BlockSpec((C, 1), lambda l: (0, 0)),
            pl.BlockSpec((C, 1), lambda l: (0, 0)),
        ],
        out_specs=pl.BlockSpec((1, 1), lambda l: (0, 0)),
        out_shape=jax.ShapeDtypeStruct((1, 1), jnp.float32),
        scratch_shapes=[
            pltpu.VMEM((128, 128), jnp.float32),
            pltpu.VMEM((128, 128), jnp.float32),
            pltpu.VMEM((128, 128), jnp.float32),
            pltpu.VMEM((128, 128), jnp.float32),
            pltpu.VMEM((128, 128), jnp.float32),
            pltpu.VMEM((128, 128), jnp.float32),
            pltpu.VMEM((2, 128), jnp.float32),
        ],
        compiler_params=pltpu.CompilerParams(
            dimension_semantics=("arbitrary",)),
    )(em_p3, lab_p3, trans, start2, end2)
    return loss


def kernel(x, labels, table, W_fc, b_fc, start_t, end_t, trans):
    B, L = x.shape
    V, D = table.shape
    C = W_fc.shape[1]
    tableT = jnp.swapaxes(table, 0, 1)
    Wpad = jnp.pad(W_fc, ((0, 0), (0, 128 - C)))
    bpad = jnp.pad(b_fc, (0, 128 - C)).reshape(1, 128)
    ptable128 = _project_tableY(tableT, Wpad, bpad)
    xT = jnp.swapaxes(x, 0, 1).reshape(-1)
    em_p = _sc_gather(ptable128, xT, C)
    em_p3 = em_p.reshape(L, B // 8, 8 * C)
    labels_T = jnp.swapaxes(labels, 0, 1)
    lab_p3 = jnp.repeat(labels_T.reshape(L, B // 8, 8), C, axis=2)
    loss = _crf_call(em_p3, lab_p3, trans,
                     start_t.reshape(C, 1), end_t.reshape(C, 1))
    logits = jnp.swapaxes(em_p.reshape(L, B, C), 0, 1)
    return (logits, loss[0, 0])
